# Initial kernel scaffold; baseline (speedup 1.0000x reference)
import jax
import jax.numpy as jnp
from jax.experimental import pallas as pl


def kernel(inputs, wordEmbed):
    raise NotImplementedError("write your pallas kernel here")


# speedup vs baseline: 3.3145x; 3.3145x over previous
import functools

import jax
import jax.numpy as jnp
from jax import lax
from jax.experimental import pallas as pl
from jax.experimental.pallas import tpu as pltpu
from jax.experimental.pallas import tpu_sc as plsc

N_WORDS = 100000
DIM = 128

NC = 2
NS = 16
NW = NC * NS

B_TOTAL = 4096 * 50
B_PER_W = B_TOTAL // NW
CHUNK = 128
NCHUNK = B_PER_W // CHUNK
NBUF = 5
NGROUP = NCHUNK // NBUF

IDX_ROWS_PER_W = B_PER_W // CHUNK


def _emb_body(idx_hbm, table_hbm, out_hbm, idx_v, rows_v, *sems):
  gsems = sems[:NBUF]
  wsems = sems[NBUF:]
  wid = lax.axis_index("s") * NC + lax.axis_index("c")
  base = wid * B_PER_W

  pltpu.sync_copy(idx_hbm.at[wid], idx_v)

  @pl.loop(0, NGROUP)
  def _group(g):
    c0 = g * NBUF
    for b in range(NBUF):
      @pl.when(g > 0)
      def _():
        pltpu.make_async_copy(
            rows_v.at[b], out_hbm.at[pl.ds(base, CHUNK)], wsems[b]).wait()
      pltpu.async_copy(table_hbm.at[idx_v.at[c0 + b]], rows_v.at[b], gsems[b])
    for b in range(NBUF):
      pltpu.make_async_copy(
          table_hbm.at[idx_v.at[c0 + b]], rows_v.at[b], gsems[b]).wait()
      pltpu.async_copy(
          rows_v.at[b], out_hbm.at[pl.ds(base + (c0 + b) * CHUNK, CHUNK)],
          wsems[b])

  for b in range(NBUF):
    pltpu.make_async_copy(
        rows_v.at[b], out_hbm.at[pl.ds(base, CHUNK)], wsems[b]).wait()


@jax.jit
def _embed(idx2d, table):
  mesh = plsc.VectorSubcoreMesh(
      core_axis_name="c", subcore_axis_name="s", num_cores=NC,
      num_subcores=NS)
  scratch = [
      pltpu.VMEM((IDX_ROWS_PER_W, CHUNK), jnp.int32),
      pltpu.VMEM((NBUF, CHUNK, DIM), jnp.float32),
  ] + [pltpu.SemaphoreType.DMA] * (2 * NBUF)
  run = pl.kernel(
      _emb_body,
      out_type=jax.ShapeDtypeStruct((B_TOTAL, DIM), jnp.float32),
      mesh=mesh,
      scratch_types=scratch,
  )
  return run(idx2d, table)


def kernel(inputs, wordEmbed):
  idx3d = inputs.reshape(NW, IDX_ROWS_PER_W, CHUNK).astype(jnp.int32)
  out = _embed(idx3d, wordEmbed)
  return out.reshape(inputs.shape[0], inputs.shape[1], DIM)


# speedup vs baseline: 5.9266x; 1.7881x over previous
import functools

import jax
import jax.numpy as jnp
from jax import lax
from jax.experimental import pallas as pl
from jax.experimental.pallas import tpu as pltpu
from jax.experimental.pallas import tpu_sc as plsc

N_WORDS = 100000
DIM = 128
BATCH = 4096
SEQ = 50

NC = 2
NS = 16
NW = NC * NS

ROWS_PER_W = BATCH // NW
RPC = 1
NCHUNK = ROWS_PER_W // RPC
NBUF = 8
NGROUP = NCHUNK // NBUF


def _emb_body(idx_hbm, table_hbm, out_hbm, idx_v, rows_v, *sems):
  gsems = sems[:NBUF]
  wsems = sems[NBUF:]
  wid = lax.axis_index("s") * NC + lax.axis_index("c")
  base = wid * ROWS_PER_W

  pltpu.sync_copy(idx_hbm.at[wid], idx_v)

  @pl.loop(0, NGROUP)
  def _group(g):
    c0 = g * NBUF
    for b in range(NBUF):
      @pl.when(g > 0)
      def _():
        pltpu.make_async_copy(
            rows_v.at[b], out_hbm.at[base], wsems[b]).wait()
      pltpu.async_copy(
          table_hbm.at[idx_v.at[c0 + b]], rows_v.at[b], gsems[b])
    for b in range(NBUF):
      pltpu.make_async_copy(
          table_hbm.at[idx_v.at[c0 + b]], rows_v.at[b], gsems[b]).wait()
      pltpu.async_copy(rows_v.at[b], out_hbm.at[base + c0 + b], wsems[b])

  for b in range(NBUF):
    pltpu.make_async_copy(
        rows_v.at[b], out_hbm.at[base], wsems[b]).wait()


@jax.jit
def _embed(idx3d, table):
  mesh = plsc.VectorSubcoreMesh(
      core_axis_name="c", subcore_axis_name="s", num_cores=NC,
      num_subcores=NS)
  scratch = [
      pltpu.VMEM((ROWS_PER_W, SEQ), jnp.int32),
      pltpu.VMEM((NBUF, SEQ, DIM), jnp.float32),
  ] + [pltpu.SemaphoreType.DMA] * (2 * NBUF)
  run = pl.kernel(
      _emb_body,
      out_type=jax.ShapeDtypeStruct((BATCH, SEQ, DIM), jnp.float32),
      mesh=mesh,
      scratch_types=scratch,
  )
  return run(idx3d, table)


def kernel(inputs, wordEmbed):
  idx3d = inputs.reshape(NW, ROWS_PER_W, SEQ).astype(jnp.int32)
  return _embed(idx3d, wordEmbed)


# speedup vs baseline: 10.3773x; 1.7510x over previous
import jax
import jax.numpy as jnp
from jax import lax
from jax.experimental import pallas as pl
from jax.experimental.pallas import tpu as pltpu
from jax.experimental.pallas import tpu_sc as plsc

N_WORDS = 100000
DIM = 128
BATCH = 4096
SEQ = 50

NC = 2
NS = 16
NW = NC * NS

ROWS_PER_W = BATCH // NW
NCHUNK = SEQ
NBUF = 5
NGROUP = NCHUNK // NBUF


def _emb_body(idx_hbm, table_hbm, out_hbm, idx_v, rows_v, *sems):
  gsems = sems[:NBUF]
  wsems = sems[NBUF:]
  wid = lax.axis_index("s") * NC + lax.axis_index("c")
  b0 = wid * ROWS_PER_W

  pltpu.sync_copy(idx_hbm.at[wid], idx_v)

  @pl.loop(0, NGROUP)
  def _group(g):
    c0 = g * NBUF
    for b in range(NBUF):
      @pl.when(g > 0)
      def _():
        pltpu.make_async_copy(
            rows_v.at[b], out_hbm.at[0, pl.ds(b0, ROWS_PER_W)],
            wsems[b]).wait()
      pltpu.async_copy(
          table_hbm.at[idx_v.at[c0 + b]], rows_v.at[b], gsems[b])
    for b in range(NBUF):
      pltpu.make_async_copy(
          table_hbm.at[idx_v.at[c0 + b]], rows_v.at[b], gsems[b]).wait()
      pltpu.async_copy(
          rows_v.at[b], out_hbm.at[c0 + b, pl.ds(b0, ROWS_PER_W)], wsems[b])

  for b in range(NBUF):
    pltpu.make_async_copy(
        rows_v.at[b], out_hbm.at[0, pl.ds(b0, ROWS_PER_W)], wsems[b]).wait()


@jax.jit
def _embed(idx3d, table):
  mesh = plsc.VectorSubcoreMesh(
      core_axis_name="c", subcore_axis_name="s", num_cores=NC,
      num_subcores=NS)
  scratch = [
      pltpu.VMEM((SEQ, ROWS_PER_W), jnp.int32),
      pltpu.VMEM((NBUF, ROWS_PER_W, DIM), jnp.float32),
  ] + [pltpu.SemaphoreType.DMA] * (2 * NBUF)
  run = pl.kernel(
      _emb_body,
      out_type=jax.ShapeDtypeStruct((SEQ, BATCH, DIM), jnp.float32),
      mesh=mesh,
      scratch_types=scratch,
  )
  return run(idx3d, table)


def kernel(inputs, wordEmbed):
  idx3d = inputs.reshape(NW, ROWS_PER_W, SEQ).transpose(0, 2, 1)
  idx3d = idx3d.astype(jnp.int32)
  out_phys = _embed(idx3d, wordEmbed)
  return out_phys.transpose(1, 0, 2)


# speedup vs baseline: 10.6613x; 1.0274x over previous
import jax
import jax.numpy as jnp
from jax import lax
from jax.experimental import pallas as pl
from jax.experimental.pallas import tpu as pltpu
from jax.experimental.pallas import tpu_sc as plsc

N_WORDS = 100000
DIM = 128
BATCH = 4096
SEQ = 50

NC = 2
NS = 16
NW = NC * NS

ROWS_PER_W = BATCH // NW
CR = 64
CPS = ROWS_PER_W // CR
NCHUNK = SEQ * CPS
NBUF = 10
NGROUP = NCHUNK // NBUF


def _emb_body(idx_hbm, table_hbm, out_hbm, idx_v, rows_v, *sems):
  gsems = sems[:NBUF]
  wsems = sems[NBUF:]
  wid = lax.axis_index("s") * NC + lax.axis_index("c")
  b0 = wid * ROWS_PER_W

  pltpu.sync_copy(idx_hbm.at[wid], idx_v)

  @pl.loop(0, NGROUP)
  def _group(g):
    c0 = g * NBUF
    for b in range(NBUF):
      @pl.when(g > 0)
      def _():
        pltpu.make_async_copy(
            rows_v.at[b], out_hbm.at[0, pl.ds(b0, CR)], wsems[b]).wait()
      c = c0 + b
      pltpu.async_copy(
          table_hbm.at[idx_v.at[c // CPS, pl.ds((c % CPS) * CR, CR)]],
          rows_v.at[b], gsems[b])
    for b in range(NBUF):
      c = c0 + b
      pltpu.make_async_copy(
          table_hbm.at[idx_v.at[c // CPS, pl.ds((c % CPS) * CR, CR)]],
          rows_v.at[b], gsems[b]).wait()
      pltpu.async_copy(
          rows_v.at[b],
          out_hbm.at[c // CPS, pl.ds(b0 + (c % CPS) * CR, CR)], wsems[b])

  for b in range(NBUF):
    pltpu.make_async_copy(
        rows_v.at[b], out_hbm.at[0, pl.ds(b0, CR)], wsems[b]).wait()


@jax.jit
def _embed(idx3d, table):
  mesh = plsc.VectorSubcoreMesh(
      core_axis_name="c", subcore_axis_name="s", num_cores=NC,
      num_subcores=NS)
  scratch = [
      pltpu.VMEM((SEQ, ROWS_PER_W), jnp.int32),
      pltpu.VMEM((NBUF, CR, DIM), jnp.float32),
  ] + [pltpu.SemaphoreType.DMA] * (2 * NBUF)
  run = pl.kernel(
      _emb_body,
      out_type=jax.ShapeDtypeStruct((SEQ, BATCH, DIM), jnp.float32),
      mesh=mesh,
      scratch_types=scratch,
  )
  return run(idx3d, table)


def kernel(inputs, wordEmbed):
  idx3d = inputs.reshape(NW, ROWS_PER_W, SEQ).transpose(0, 2, 1)
  idx3d = idx3d.astype(jnp.int32)
  out_phys = _embed(idx3d, wordEmbed)
  return out_phys.transpose(1, 0, 2)
